# baseline (speedup 1.0000x reference)
import functools

import jax
import jax.numpy as jnp
from jax import lax
from jax.experimental import pallas as pl
from jax.experimental.pallas import tpu as pltpu
from jax.experimental.pallas import tpu_sc as plsc

_B = 4096
_S = 200
_D = 64
_NW = 32
_NB = 4
_R = _S // _NB


def _body(idx4_hbm, table_hbm, o1_hbm, o2_hbm, idx_v, rows, trans,
          g0, g1, g2, g3, w0, w1, w2, w3):
    gsem = (g0, g1, g2, g3)
    wsem = (w0, w1, w2, w3)
    wid = lax.axis_index("s") * 2 + lax.axis_index("c")

    pltpu.sync_copy(idx4_hbm.at[:, wid], idx_v)

    def start_gather(c, b):
        pltpu.async_copy(table_hbm.at[idx_v.at[c >> 3, c & 7]],
                         rows.at[b], gsem[b])

    def wait_gather(b):
        pltpu.make_async_copy(table_hbm.at[idx_v.at[0, 0]], rows.at[b],
                              gsem[b]).wait()

    def transpose_chunk(b):
        lane = lax.iota(jnp.int32, 16)
        lanes_l0 = [lane + l0 for l0 in range(0, 128, 16)]
        rows_b = rows.at[b]
        trans_b = trans.at[b]

        def do_d0(i4, _2):
            d0 = i4 * 16

            def do_rot(r2, _3):
                for half in range(2):
                    rot = r2 * 2 + half
                    d_idx = ((lane + rot) & 15) + d0
                    i_idx = d_idx >> 3
                    m_idx = d_idx & 7
                    for l_idx in lanes_l0:
                        v = plsc.load_gather(rows_b, [l_idx, d_idx])
                        plsc.store_scatter(trans_b, [i_idx, m_idx, l_idx], v)
                return _3
            return lax.fori_loop(0, 8, do_rot, _2)
        lax.fori_loop(0, 4, do_d0, 0)

    def start_write(c, b):
        pltpu.async_copy(trans.at[b], o1_hbm.at[c, :, wid], wsem[b])
        pltpu.async_copy(trans.at[b], o2_hbm.at[c, :, wid], wsem[b])

    def wait_write(b):
        pltpu.make_async_copy(trans.at[b], o1_hbm.at[0, :, wid],
                              wsem[b]).wait()
        pltpu.make_async_copy(trans.at[b], o2_hbm.at[0, :, wid],
                              wsem[b]).wait()

    for b in range(_NB):
        start_gather(b, b)
    for b in range(_NB):
        wait_gather(b)
        transpose_chunk(b)
        start_gather(b + _NB, b)
        start_write(b, b)

    def rnd(i, carry):
        for b in range(_NB):
            c = i * _NB + b
            wait_gather(b)
            wait_write(b)
            transpose_chunk(b)
            start_gather(c + _NB, b)
            start_write(c, b)
        return carry

    lax.fori_loop(1, _R - 1, rnd, 0)

    for b in range(_NB):
        c = (_R - 1) * _NB + b
        wait_gather(b)
        wait_write(b)
        transpose_chunk(b)
        start_write(c, b)
    for b in range(_NB):
        wait_write(b)


_gather = functools.partial(
    pl.kernel,
    out_type=(jax.ShapeDtypeStruct((_S, 8, 32, 8, 128), jnp.float32),
              jax.ShapeDtypeStruct((_S, 8, 32, 8, 128), jnp.float32)),
    mesh=plsc.VectorSubcoreMesh(core_axis_name="c", subcore_axis_name="s"),
    compiler_params=pltpu.CompilerParams(use_tc_tiling_on_sc=False,
                                         needs_layout_passes=False),
    scratch_types=(
        [pltpu.VMEM((_S // 8, 8, 128), jnp.int32),
         pltpu.VMEM((_NB, 128, _D), jnp.float32),
         pltpu.VMEM((_NB, 8, 8, 128), jnp.float32)]
        + [pltpu.SemaphoreType.DMA] * (2 * _NB)
    ),
)(_body)


def _mask_body(idx_ref, mask_ref):
    mask_ref[...] = (idx_ref[...] != 0).astype(jnp.uint8)


def _to_out(x5):
    y = jnp.transpose(x5, (2, 4, 0, 1, 3))
    return y.reshape(_B, _S, _D)


def kernel(input_var, W):
    idxT = input_var.T
    idx4 = idxT.reshape(_S // 8, 8, _NW, 128).transpose(0, 2, 1, 3)
    o1, o2 = _gather(idx4, W)
    maskT = pl.pallas_call(
        _mask_body,
        out_shape=jax.ShapeDtypeStruct((_S, _B), jnp.uint8),
    )(idxT)
    return (_to_out(o1), _to_out(o2), maskT.T)


# speedup vs baseline: 1.0004x; 1.0004x over previous
import functools

import jax
import jax.numpy as jnp
from jax import lax
from jax.experimental import pallas as pl
from jax.experimental.pallas import tpu as pltpu
from jax.experimental.pallas import tpu_sc as plsc

_B = 4096
_S = 200
_D = 64
_NW = 32
_NB = 5
_R = _S // _NB


def _body(idx4_hbm, table_hbm, o1_hbm, o2_hbm, idx_v, rows, trans,
          g0, g1, g2, g3, g4, w0, w1, w2, w3, w4):
    gsem = (g0, g1, g2, g3, g4)
    wsem = (w0, w1, w2, w3, w4)
    wid = lax.axis_index("s") * 2 + lax.axis_index("c")

    pltpu.sync_copy(idx4_hbm.at[:, wid], idx_v)

    def start_gather(c, b):
        pltpu.async_copy(table_hbm.at[idx_v.at[c >> 3, c & 7]],
                         rows.at[b], gsem[b])

    def wait_gather(b):
        pltpu.make_async_copy(table_hbm.at[idx_v.at[0, 0]], rows.at[b],
                              gsem[b]).wait()

    def transpose_chunk(b):
        lane = lax.iota(jnp.int32, 16)
        lanes_l0 = [lane + l0 for l0 in range(0, 128, 16)]
        rows_b = rows.at[b]
        trans_b = trans.at[b]

        def do_d0(i4, _2):
            d0 = i4 * 16

            def do_rot(r2, _3):
                for half in range(2):
                    rot = r2 * 2 + half
                    d_idx = ((lane + rot) & 15) + d0
                    i_idx = d_idx >> 3
                    m_idx = d_idx & 7
                    for l_idx in lanes_l0:
                        v = plsc.load_gather(rows_b, [l_idx, d_idx])
                        plsc.store_scatter(trans_b, [i_idx, m_idx, l_idx], v)
                return _3
            return lax.fori_loop(0, 8, do_rot, _2)
        lax.fori_loop(0, 4, do_d0, 0)

    def start_write(c, b):
        pltpu.async_copy(trans.at[b], o1_hbm.at[c, :, wid], wsem[b])
        pltpu.async_copy(trans.at[b], o2_hbm.at[c, :, wid], wsem[b])

    def wait_write(b):
        pltpu.make_async_copy(trans.at[b], o1_hbm.at[0, :, wid],
                              wsem[b]).wait()
        pltpu.make_async_copy(trans.at[b], o2_hbm.at[0, :, wid],
                              wsem[b]).wait()

    for b in range(_NB):
        start_gather(b, b)
    for b in range(_NB):
        wait_gather(b)
        transpose_chunk(b)
        start_gather(b + _NB, b)
        start_write(b, b)

    def rnd(i, carry):
        for b in range(_NB):
            c = i * _NB + b
            wait_gather(b)
            wait_write(b)
            transpose_chunk(b)
            start_gather(c + _NB, b)
            start_write(c, b)
        return carry

    lax.fori_loop(1, _R - 1, rnd, 0)

    for b in range(_NB):
        c = (_R - 1) * _NB + b
        wait_gather(b)
        wait_write(b)
        transpose_chunk(b)
        start_write(c, b)
    for b in range(_NB):
        wait_write(b)


_gather = functools.partial(
    pl.kernel,
    out_type=(jax.ShapeDtypeStruct((_S, 8, 32, 8, 128), jnp.float32),
              jax.ShapeDtypeStruct((_S, 8, 32, 8, 128), jnp.float32)),
    mesh=plsc.VectorSubcoreMesh(core_axis_name="c", subcore_axis_name="s"),
    compiler_params=pltpu.CompilerParams(use_tc_tiling_on_sc=False,
                                         needs_layout_passes=False),
    scratch_types=(
        [pltpu.VMEM((_S // 8, 8, 128), jnp.int32),
         pltpu.VMEM((_NB, 128, _D), jnp.float32),
         pltpu.VMEM((_NB, 8, 8, 128), jnp.float32)]
        + [pltpu.SemaphoreType.DMA] * (2 * _NB)
    ),
)(_body)


def _mask_body(idx_ref, mask_ref):
    mask_ref[...] = (idx_ref[...] != 0).astype(jnp.uint8)


def _to_out(x5):
    y = jnp.transpose(x5, (2, 4, 0, 1, 3))
    return y.reshape(_B, _S, _D)


def kernel(input_var, W):
    idxT = input_var.T
    idx4 = idxT.reshape(_S // 8, 8, _NW, 128).transpose(0, 2, 1, 3)
    o1, o2 = _gather(idx4, W)
    maskT = pl.pallas_call(
        _mask_body,
        out_shape=jax.ShapeDtypeStruct((_S, _B), jnp.uint8),
    )(idxT)
    return (_to_out(o1), _to_out(o2), maskT.T)


# speedup vs baseline: 1.2074x; 1.2069x over previous
import functools

import jax
import jax.numpy as jnp
from jax import lax
from jax.experimental import pallas as pl
from jax.experimental.pallas import tpu as pltpu
from jax.experimental.pallas import tpu_sc as plsc

_B = 4096
_S = 200
_D = 64
_NW = 32
_NB = 5
_R = _S // _NB


def _body(idx4_hbm, table_hbm, o1_hbm, o2_hbm, idx_v, rows, trans,
          g0, g1, g2, g3, g4, w0, w1, w2, w3, w4):
    gsem = (g0, g1, g2, g3, g4)
    wsem = (w0, w1, w2, w3, w4)
    wid = lax.axis_index("s") * 2 + lax.axis_index("c")

    pltpu.sync_copy(idx4_hbm.at[:, wid], idx_v)

    def start_gather(c, b):
        pltpu.async_copy(table_hbm.at[idx_v.at[c >> 3, c & 7]],
                         rows.at[b], gsem[b])

    def wait_gather(b):
        pltpu.make_async_copy(table_hbm.at[idx_v.at[0, 0]], rows.at[b],
                              gsem[b]).wait()

    def transpose_chunk(b):
        lane = lax.iota(jnp.int32, 16)
        lanes_l0 = [lane + l0 for l0 in range(0, 128, 16)]
        rows_b = rows.at[b]
        trans_b = trans.at[b]

        def do_d0(i4, _2):
            d0 = i4 * 16

            @plsc.parallel_loop(0, 16, unroll=2)
            def do_rot(rot):
                d_idx = ((lane + rot) & 15) + d0
                i_idx = d_idx >> 3
                m_idx = d_idx & 7
                for l_idx in lanes_l0:
                    v = plsc.load_gather(rows_b, [l_idx, d_idx])
                    plsc.store_scatter(trans_b, [i_idx, m_idx, l_idx], v)
            return _2
        lax.fori_loop(0, 4, do_d0, 0)

    def start_write(c, b):
        pltpu.async_copy(trans.at[b], o1_hbm.at[c, :, wid], wsem[b])
        pltpu.async_copy(trans.at[b], o2_hbm.at[c, :, wid], wsem[b])

    def wait_write(b):
        pltpu.make_async_copy(trans.at[b], o1_hbm.at[0, :, wid],
                              wsem[b]).wait()
        pltpu.make_async_copy(trans.at[b], o2_hbm.at[0, :, wid],
                              wsem[b]).wait()

    for b in range(_NB):
        start_gather(b, b)
    for b in range(_NB):
        wait_gather(b)
        transpose_chunk(b)
        start_gather(b + _NB, b)
        start_write(b, b)

    def rnd(i, carry):
        for b in range(_NB):
            c = i * _NB + b
            wait_gather(b)
            wait_write(b)
            transpose_chunk(b)
            start_gather(c + _NB, b)
            start_write(c, b)
        return carry

    lax.fori_loop(1, _R - 1, rnd, 0)

    for b in range(_NB):
        c = (_R - 1) * _NB + b
        wait_gather(b)
        wait_write(b)
        transpose_chunk(b)
        start_write(c, b)
    for b in range(_NB):
        wait_write(b)


_gather = functools.partial(
    pl.kernel,
    out_type=(jax.ShapeDtypeStruct((_S, 8, 32, 8, 128), jnp.float32),
              jax.ShapeDtypeStruct((_S, 8, 32, 8, 128), jnp.float32)),
    mesh=plsc.VectorSubcoreMesh(core_axis_name="c", subcore_axis_name="s"),
    compiler_params=pltpu.CompilerParams(use_tc_tiling_on_sc=False,
                                         needs_layout_passes=False),
    scratch_types=(
        [pltpu.VMEM((_S // 8, 8, 128), jnp.int32),
         pltpu.VMEM((_NB, 128, _D), jnp.float32),
         pltpu.VMEM((_NB, 8, 8, 128), jnp.float32)]
        + [pltpu.SemaphoreType.DMA] * (2 * _NB)
    ),
)(_body)


def _mask_body(idx_ref, mask_ref):
    mask_ref[...] = (idx_ref[...] != 0).astype(jnp.uint8)


def _to_out(x5):
    y = jnp.transpose(x5, (2, 4, 0, 1, 3))
    return y.reshape(_B, _S, _D)


def kernel(input_var, W):
    idxT = input_var.T
    idx4 = idxT.reshape(_S // 8, 8, _NW, 128).transpose(0, 2, 1, 3)
    o1, o2 = _gather(idx4, W)
    maskT = pl.pallas_call(
        _mask_body,
        out_shape=jax.ShapeDtypeStruct((_S, _B), jnp.uint8),
    )(idxT)
    return (_to_out(o1), _to_out(o2), maskT.T)
